# Initial kernel scaffold; baseline (speedup 1.0000x reference)
import jax
import jax.numpy as jnp
from jax.experimental import pallas as pl


def kernel(hidden_states, gate_w, w_gate, w_up, w_down, ws_gate, ws_up, ws_down):
    raise NotImplementedError("write your pallas kernel here")


# speedup vs baseline: 1.6203x; 1.6203x over previous
import functools

import jax
import jax.numpy as jnp
from jax.experimental import pallas as pl

D_MODEL = 1024
MOE_FF = 512
SHARED_FF = 1024
N_EXPERTS = 8
TOP_K = 2

TILE_T = 256


def _moe_kernel(x_ref, gate_w_ref, wg_ref, wu_ref, wd_ref,
                wsg_ref, wsu_ref, wsd_ref, out_ref):
    x = x_ref[...]
    xb = x.astype(jnp.bfloat16)

    logits = jnp.dot(x, gate_w_ref[...].T,
                     preferred_element_type=jnp.float32)
    scores = jax.nn.softmax(logits, axis=-1)
    w1 = jnp.max(scores, axis=-1, keepdims=True)
    a1 = jnp.argmax(scores, axis=-1)
    e_iota = jax.lax.broadcasted_iota(jnp.int32, scores.shape, 1)
    masked = jnp.where(e_iota == a1[:, None], -jnp.inf, scores)
    w2 = jnp.max(masked, axis=-1, keepdims=True)
    a2 = jnp.argmax(masked, axis=-1)
    denom = w1 + w2 + 1e-20
    c1 = w1 / denom
    c2 = w2 / denom
    combine = (jnp.where(e_iota == a1[:, None], c1, 0.0)
               + jnp.where(e_iota == a2[:, None], c2, 0.0))

    acc = jnp.zeros(x.shape, dtype=jnp.float32)
    for e in range(N_EXPERTS):
        g = jnp.dot(xb, wg_ref[e].T, preferred_element_type=jnp.float32)
        u = jnp.dot(xb, wu_ref[e].T, preferred_element_type=jnp.float32)
        h = (jax.nn.silu(g) * u).astype(jnp.bfloat16)
        o = jnp.dot(h, wd_ref[e].T, preferred_element_type=jnp.float32)
        acc = acc + combine[:, e][:, None] * o

    gs = jnp.dot(xb, wsg_ref[...].T, preferred_element_type=jnp.float32)
    us = jnp.dot(xb, wsu_ref[...].T, preferred_element_type=jnp.float32)
    hs = (jax.nn.silu(gs) * us).astype(jnp.bfloat16)
    acc = acc + jnp.dot(hs, wsd_ref[...].T, preferred_element_type=jnp.float32)

    out_ref[...] = acc


@functools.partial(jax.jit, static_argnames=())
def kernel(hidden_states, gate_w, w_gate, w_up, w_down, ws_gate, ws_up, ws_down):
    orig_shape = hidden_states.shape
    x = hidden_states.reshape(-1, orig_shape[-1])
    T, D = x.shape

    wg = w_gate.astype(jnp.bfloat16)
    wu = w_up.astype(jnp.bfloat16)
    wd = w_down.astype(jnp.bfloat16)
    wsg = ws_gate.astype(jnp.bfloat16)
    wsu = ws_up.astype(jnp.bfloat16)
    wsd = ws_down.astype(jnp.bfloat16)

    n_tiles = T // TILE_T
    full = lambda shape: pl.BlockSpec(shape, lambda i: (0,) * len(shape))

    out = pl.pallas_call(
        _moe_kernel,
        grid=(n_tiles,),
        in_specs=[
            pl.BlockSpec((TILE_T, D), lambda i: (i, 0)),
            full((N_EXPERTS, D)),
            full((N_EXPERTS, MOE_FF, D)),
            full((N_EXPERTS, MOE_FF, D)),
            full((N_EXPERTS, D, MOE_FF)),
            full((SHARED_FF, D)),
            full((SHARED_FF, D)),
            full((D, SHARED_FF)),
        ],
        out_specs=pl.BlockSpec((TILE_T, D), lambda i: (i, 0)),
        out_shape=jax.ShapeDtypeStruct((T, D), jnp.float32),
    )(x, gate_w, wg, wu, wd, wsg, wsu, wsd)

    return out.reshape(orig_shape)


# speedup vs baseline: 1.7230x; 1.0634x over previous
import functools

import jax
import jax.numpy as jnp
from jax.experimental import pallas as pl

D_MODEL = 1024
MOE_FF = 512
SHARED_FF = 1024
N_EXPERTS = 8
TOP_K = 2

TILE_T = 512


def _moe_kernel(x_ref, gate_w_ref, wg_ref, wu_ref, wd_ref,
                wsg_ref, wsu_ref, wsd_ref, out_ref):
    x = x_ref[...]
    xb = x.astype(jnp.bfloat16)

    logits = jnp.dot(x, gate_w_ref[...].T,
                     preferred_element_type=jnp.float32)
    scores = jax.nn.softmax(logits, axis=-1)
    w1 = jnp.max(scores, axis=-1, keepdims=True)
    a1 = jnp.argmax(scores, axis=-1)
    e_iota = jax.lax.broadcasted_iota(jnp.int32, scores.shape, 1)
    masked = jnp.where(e_iota == a1[:, None], -jnp.inf, scores)
    w2 = jnp.max(masked, axis=-1, keepdims=True)
    a2 = jnp.argmax(masked, axis=-1)
    denom = w1 + w2 + 1e-20
    c1 = w1 / denom
    c2 = w2 / denom
    combine = (jnp.where(e_iota == a1[:, None], c1, 0.0)
               + jnp.where(e_iota == a2[:, None], c2, 0.0))

    acc = jnp.zeros(x.shape, dtype=jnp.float32)
    for e in range(N_EXPERTS):
        g = jnp.dot(xb, wg_ref[e].T, preferred_element_type=jnp.float32)
        u = jnp.dot(xb, wu_ref[e].T, preferred_element_type=jnp.float32)
        h = (jax.nn.silu(g) * u).astype(jnp.bfloat16)
        o = jnp.dot(h, wd_ref[e].T, preferred_element_type=jnp.float32)
        acc = acc + combine[:, e][:, None] * o

    gs = jnp.dot(xb, wsg_ref[...].T, preferred_element_type=jnp.float32)
    us = jnp.dot(xb, wsu_ref[...].T, preferred_element_type=jnp.float32)
    hs = (jax.nn.silu(gs) * us).astype(jnp.bfloat16)
    acc = acc + jnp.dot(hs, wsd_ref[...].T, preferred_element_type=jnp.float32)

    out_ref[...] = acc


@functools.partial(jax.jit, static_argnames=())
def kernel(hidden_states, gate_w, w_gate, w_up, w_down, ws_gate, ws_up, ws_down):
    orig_shape = hidden_states.shape
    x = hidden_states.reshape(-1, orig_shape[-1])
    T, D = x.shape

    wg = w_gate.astype(jnp.bfloat16)
    wu = w_up.astype(jnp.bfloat16)
    wd = w_down.astype(jnp.bfloat16)
    wsg = ws_gate.astype(jnp.bfloat16)
    wsu = ws_up.astype(jnp.bfloat16)
    wsd = ws_down.astype(jnp.bfloat16)

    n_tiles = T // TILE_T
    full = lambda shape: pl.BlockSpec(shape, lambda i: (0,) * len(shape))

    out = pl.pallas_call(
        _moe_kernel,
        grid=(n_tiles,),
        in_specs=[
            pl.BlockSpec((TILE_T, D), lambda i: (i, 0)),
            full((N_EXPERTS, D)),
            full((N_EXPERTS, MOE_FF, D)),
            full((N_EXPERTS, MOE_FF, D)),
            full((N_EXPERTS, D, MOE_FF)),
            full((SHARED_FF, D)),
            full((SHARED_FF, D)),
            full((D, SHARED_FF)),
        ],
        out_specs=pl.BlockSpec((TILE_T, D), lambda i: (i, 0)),
        out_shape=jax.ShapeDtypeStruct((T, D), jnp.float32),
    )(x, gate_w, wg, wu, wd, wsg, wsu, wsd)

    return out.reshape(orig_shape)
